# baseline (speedup 1.0000x reference)
import functools

import jax
import jax.numpy as jnp
from jax import lax
from jax.experimental import pallas as pl
from jax.experimental.pallas import tpu as pltpu
from jax.experimental.pallas import tpu_sc as plsc

N = 2
C = 128
DIM = 48
P = 100000
NW = 32
CH = 32
CPW = 198
PW = CH * CPW
TOTAL = NW * PW
R = N * DIM ** 3
SCALE = DIM / (DIM - 1.0)
L = 16


def _axis_interp(coord):
    pos = coord * SCALE - 0.5
    i0 = (pos + 1.0).astype(jnp.int32) - 1
    frac = pos - i0.astype(jnp.float32)
    w1 = frac
    w0 = 1.0 - frac
    w0 = jnp.where(i0 >= 0, w0, 0.0)
    w1 = jnp.where(i0 < DIM - 1, w1, 0.0)
    i0c = jnp.maximum(i0, 0)
    i1c = jnp.minimum(i0 + 1, DIM - 1)
    return (i0c, i1c), (w0, w1)


mesh = plsc.VectorSubcoreMesh(core_axis_name="c", subcore_axis_name="s")


@functools.partial(
    pl.kernel,
    mesh=mesh,
    out_type=jax.ShapeDtypeStruct((TOTAL, C), jnp.float32),
    scratch_types=[
        pltpu.VMEM((4, PW), jnp.float32),
        pltpu.VMEM((2, 8, CH), jnp.int32),
        pltpu.VMEM((2, 8, CH), jnp.float32),
        pltpu.VMEM((2, 8, CH, C), jnp.float32),
        pltpu.VMEM((2, CH, C), jnp.float32),
        pltpu.SemaphoreType.DMA,
        pltpu.SemaphoreType.DMA,
        pltpu.SemaphoreType.DMA,
        pltpu.SemaphoreType.DMA,
    ],
)
def _interp_kernel(table_hbm, coords_hbm, out_hbm,
                   coords_v, idx_v, w_v, rows_v, out_v,
                   gsem0, gsem1, osem0, osem1):
    wid = lax.axis_index("s") * 2 + lax.axis_index("c")
    gsems = (gsem0, gsem1)
    osems = (osem0, osem1)

    pltpu.sync_copy(coords_hbm.at[wid], coords_v)

    def prep(i, b):
        offp = i * CH
        for g in range(CH // L):
            s = offp + g * L
            x = coords_v[0, pl.ds(s, L)]
            y = coords_v[1, pl.ds(s, L)]
            z = coords_v[2, pl.ds(s, L)]
            bas = coords_v[3, pl.ds(s, L)].astype(jnp.int32)
            xi, xw = _axis_interp(x)
            yi, yw = _axis_interp(y)
            zi, zw = _axis_interp(z)
            cidx = 0
            for dz in range(2):
                zoff = bas + zi[dz] * (DIM * DIM)
                for dy in range(2):
                    yzoff = zoff + yi[dy] * DIM
                    yzw = zw[dz] * yw[dy]
                    for dx in range(2):
                        idx_v[b, cidx, pl.ds(g * L, L)] = yzoff + xi[dx]
                        w_v[b, cidx, pl.ds(g * L, L)] = yzw * xw[dx]
                        cidx += 1
        for cc in range(8):
            pltpu.make_async_copy(
                table_hbm.at[idx_v.at[b, cc]], rows_v.at[b, cc], gsems[b]
            ).start()

    def wait_gathers(b):
        for cc in range(8):
            pltpu.make_async_copy(
                table_hbm.at[idx_v.at[b, cc]], rows_v.at[b, cc], gsems[b]
            ).wait()

    def wsum(b):
        def group(g, carry):
            s = g * L
            wvecs = [w_v[b, cc, pl.ds(s, L)] for cc in range(8)]
            for j in range(L):
                k = s + j
                accs = None
                for cc in range(8):
                    w = wvecs[cc][j]
                    if accs is None:
                        accs = [rows_v[b, cc, k, pl.ds(v * L, L)] * w
                                for v in range(C // L)]
                    else:
                        for v in range(C // L):
                            accs[v] = (accs[v]
                                       + rows_v[b, cc, k, pl.ds(v * L, L)] * w)
                for v in range(C // L):
                    out_v[b, k, pl.ds(v * L, L)] = accs[v]
            return carry
        lax.fori_loop(0, CH // L, group, 0, unroll=False)

    def fire_out(i, b):
        row0 = wid * PW + i * CH
        pltpu.make_async_copy(
            out_v.at[b], out_hbm.at[pl.ds(row0, CH)], osems[b]
        ).start()

    def wait_out(i, b):
        row0 = wid * PW + i * CH
        pltpu.make_async_copy(
            out_v.at[b], out_hbm.at[pl.ds(row0, CH)], osems[b]
        ).wait()

    prep(0, 0)
    prep(1, 1)

    def pair_body(t, carry):
        for b in range(2):
            i = 2 * t + b
            wait_gathers(b)

            @pl.when(i >= 2)
            def _():
                wait_out(i - 2, b)

            wsum(b)
            fire_out(i, b)

            @pl.when(i + 2 < CPW)
            def _():
                prep(i + 2, b)
        return carry

    lax.fori_loop(0, CPW // 2, pair_body, 0)

    wait_out(CPW - 2, 0)
    wait_out(CPW - 1, 1)


def kernel(encoder_outputs, graph_coords):
    table = jnp.transpose(encoder_outputs, (0, 2, 3, 4, 1)).reshape(R, C)
    coords = graph_coords.reshape(N * P, 3)
    pad = TOTAL - N * P
    coords = jnp.concatenate(
        [coords, jnp.zeros((pad, 3), coords.dtype)], axis=0)
    base = (jnp.arange(TOTAL) >= P).astype(jnp.float32) * float(DIM ** 3)
    packed = jnp.concatenate([coords, base[:, None]], axis=1)
    coords_w = packed.T.reshape(4, NW, PW).transpose(1, 0, 2)
    out = _interp_kernel(table, coords_w)
    return out[: N * P].reshape(N, 1, 1, P, C)
